# Initial kernel scaffold; baseline (speedup 1.0000x reference)
import jax
import jax.numpy as jnp
from jax.experimental import pallas as pl


def kernel(x, ln_scale, ln_bias, shared_in_w, shared_out_w, shared_out_b, expert_in_w, expert_out_w, expert_out_b, group_gate_w, expert_gate_w, group_bias_buf, expert_bias_buf):
    raise NotImplementedError("write your pallas kernel here")


# speedup vs baseline: 13.1364x; 13.1364x over previous
import functools

import jax
import jax.numpy as jnp
from jax.experimental import pallas as pl
from jax.experimental.pallas import tpu as pltpu

B, T, C, H = 1, 512, 1024, 512
G, EPG, K = 8, 8, 2
E = G * EPG
S = B * T
NEG = -1e30


def _silu(x):
    return x * jax.nn.sigmoid(x)


def _routing_kernel(x_ref, ln_scale_ref, ln_bias_ref, shared_in_ref,
                    shared_out_ref, shared_out_b_ref, expert_in_ref,
                    expert_out_b_ref, group_gate_ref, expert_gate_ref,
                    group_bias_ref, expert_bias_ref,
                    base_ref, h_ref, coef_ref):
    flat = x_ref[...]
    mu = jnp.mean(flat, axis=-1, keepdims=True)
    var = jnp.mean((flat - mu) ** 2, axis=-1, keepdims=True)
    flat = (flat - mu) * jax.lax.rsqrt(var + 1e-5)
    flat = flat * ln_scale_ref[...] + ln_bias_ref[...]

    hs = jnp.dot(flat, shared_in_ref[...], preferred_element_type=jnp.float32)
    a = hs[:, :H]
    b = hs[:, H:]
    h_shared = _silu(a) * b
    out_shared = (jnp.dot(h_shared, shared_out_ref[...],
                          preferred_element_type=jnp.float32)
                  + shared_out_b_ref[...])

    g_logits = (jnp.dot(flat, group_gate_ref[...],
                        preferred_element_type=jnp.float32)
                + group_bias_ref[...])
    g_max = jnp.max(g_logits, axis=-1, keepdims=True)
    g_iota = jax.lax.broadcasted_iota(jnp.int32, (S, G), 1)
    group_idx = jnp.min(jnp.where(g_logits == g_max, g_iota, G),
                        axis=-1, keepdims=True)

    e_logits = (jnp.dot(flat, expert_gate_ref[...],
                        preferred_element_type=jnp.float32)
                + expert_bias_ref[...])
    e_iota = jax.lax.broadcasted_iota(jnp.int32, (S, E), 1)
    in_group = (e_iota // EPG) == group_idx
    e_masked = jnp.where(in_group, e_logits, NEG)
    m = jnp.max(e_masked, axis=-1, keepdims=True)
    p = jnp.exp(e_masked - m)
    p = p / jnp.sum(p, axis=-1, keepdims=True)

    v1 = jnp.max(p, axis=-1, keepdims=True)
    i1 = jnp.min(jnp.where(p == v1, e_iota, E), axis=-1, keepdims=True)
    p2 = jnp.where(e_iota == i1, -1.0, p)
    v2 = jnp.max(p2, axis=-1, keepdims=True)
    i2 = jnp.min(jnp.where(p2 == v2, e_iota, E), axis=-1, keepdims=True)
    denom = v1 + v2 + 1e-8
    w1 = v1 / denom
    w2 = v2 / denom

    sel1 = (e_iota == i1).astype(jnp.float32)
    sel2 = (e_iota == i2).astype(jnp.float32)
    coef = sel1 * w1 + sel2 * w2
    coef_ref[...] = coef

    bias_routed = jnp.dot(sel1 + sel2, expert_out_b_ref[...],
                          preferred_element_type=jnp.float32)
    base_ref[...] = out_shared + bias_routed

    he = jnp.dot(flat, expert_in_ref[...], preferred_element_type=jnp.float32)
    h_ref[...] = _silu(he[:, :H]) * he[:, H:]


def _expert_kernel(base_ref, h_ref, coef_ref, w2_ref, out_ref):
    e = pl.program_id(0)
    onehot = (jax.lax.broadcasted_iota(jnp.int32, (E, 1), 0) == e
              ).astype(jnp.float32)
    c = jnp.dot(coef_ref[...], onehot, preferred_element_type=jnp.float32)
    scaled = h_ref[...] * c
    contrib = jax.lax.dot_general(
        scaled, w2_ref[0],
        dimension_numbers=(((1,), (1,)), ((), ())),
        preferred_element_type=jnp.float32)

    @pl.when(e == 0)
    def _():
        out_ref[...] = base_ref[...] + contrib

    @pl.when(e > 0)
    def _():
        out_ref[...] = out_ref[...] + contrib


def kernel(x, ln_scale, ln_bias, shared_in_w, shared_out_w, shared_out_b,
           expert_in_w, expert_out_w, expert_out_b, group_gate_w,
           expert_gate_w, group_bias_buf, expert_bias_buf):
    flat = x.reshape(S, C)

    base, h_expert, coef = pl.pallas_call(
        _routing_kernel,
        out_shape=[
            jax.ShapeDtypeStruct((S, C), jnp.float32),
            jax.ShapeDtypeStruct((S, H), jnp.float32),
            jax.ShapeDtypeStruct((S, E), jnp.float32),
        ],
    )(flat, ln_scale.reshape(1, C), ln_bias.reshape(1, C), shared_in_w,
      shared_out_w, shared_out_b.reshape(1, C), expert_in_w, expert_out_b,
      group_gate_w, expert_gate_w, group_bias_buf.reshape(1, G),
      expert_bias_buf.reshape(1, E))

    out = pl.pallas_call(
        _expert_kernel,
        grid=(E,),
        in_specs=[
            pl.BlockSpec((S, C), lambda e: (0, 0)),
            pl.BlockSpec((S, H), lambda e: (0, 0)),
            pl.BlockSpec((S, E), lambda e: (0, 0)),
            pl.BlockSpec((1, C, H), lambda e: (e, 0, 0)),
        ],
        out_specs=pl.BlockSpec((S, C), lambda e: (0, 0)),
        out_shape=jax.ShapeDtypeStruct((S, C), jnp.float32),
    )(base, h_expert, coef, expert_out_w)

    return out.reshape(B, T, C)
